# baseline (speedup 1.0000x reference)
import jax
import jax.numpy as jnp
import numpy as np
from jax.experimental import pallas as pl
from jax.experimental.pallas import tpu as pltpu

_B = 2
_C = 96
_H = 224
_W = 224
_SL = 64
_R = 8
_HS = 56
_WS = 56

_TH = 32
_NT = _H // _TH
_NFLAT = _TH * _W
_UROWS = _TH // 4 + 2
_UFLAT = (_UROWS * 4) * _W


def _stats_kernel(fp_ref, out_ref):
    b = pl.program_id(0)
    t = pl.program_id(1)

    @pl.when(jnp.logical_and(b == 0, t == 0))
    def _init():
        out_ref[...] = jnp.zeros_like(out_ref)

    x = fp_ref[0]
    out_ref[0, :] += jnp.sum(x, axis=(1, 2))
    out_ref[1, :] += jnp.sum(x * x, axis=(1, 2))


def _main_kernel(fp_ref, sg_ref, gcat_ref, uh_ref, mk_ref, out_ref,
                 f_ref, oc_ref):
    b = pl.program_id(0)
    t = pl.program_id(1)

    @pl.when(jnp.logical_and(b == 0, t == 0))
    def _init_f():
        f_ref[81, :] = jnp.ones((_NFLAT,), jnp.float32)
        f_ref[82:128, :] = jnp.zeros((46, _NFLAT), jnp.float32)
        oc_ref[:, 0, :] = jnp.zeros((_R + 1, _W), jnp.float32)
        oc_ref[:, _HS + 1, :] = jnp.zeros((_R + 1, _W), jnp.float32)

    @pl.when(t == 0)
    def _build_onehot():
        uh = uh_ref[...]
        na = jnp.ones((_HS, _WS), jnp.float32)
        for j in range(_R - 1, -1, -1):
            mj = jnp.where(sg_ref[0, j] != 0.0, 1.0, 0.0)
            oj = mj * na
            na = na * (1.0 - mj)
            oc_ref[j, 1:_HS + 1, :] = jnp.dot(
                oj, uh, preferred_element_type=jnp.float32)
        oc_ref[_R, 1:_HS + 1, :] = jnp.dot(
            na, uh, preferred_element_type=jnp.float32)

    r10 = oc_ref[:, pl.ds((_TH // 4) * t, _UROWS), :]
    r40 = jnp.broadcast_to(r10[:, :, None, :], (_R + 1, _UROWS, 4, _W))
    uflat = r40.reshape(_R + 1, _UFLAT)
    base = 4 * _W
    mkl = mk_ref[0]
    mkr = mk_ref[1]
    for dy in range(3):
        for dx in range(3):
            tap = dy * 3 + dx
            st = base + (dy - 1) * _W + (dx - 1)
            slab = uflat[:, st:st + _NFLAT]
            if dx == 0:
                slab = slab * mkl[None, :]
            elif dx == 2:
                slab = slab * mkr[None, :]
            f_ref[tap * 9:(tap + 1) * 9, :] = slab

    gb = jnp.dot(gcat_ref[0], f_ref[...],
                 preferred_element_type=jnp.float32)
    x = fp_ref[0]
    out_ref[0] = x * gb[:_C] + gb[_C:]


def _build_tables(sg, style_codes, mask_codes, bn_w, bn_b, fc_w, fc_b,
                  gamma_w, gamma_b, beta_w, beta_b, mean, var):
    use = (mask_codes[:, :_R] == 1)[:, :, None]
    codes = jnp.where(use, style_codes[:, :_R], style_codes[:, _R:_R + 1])
    mu = jax.nn.relu(jnp.einsum('brs,rts->brt', codes, fc_w) + fc_b[None])
    mu9 = jnp.concatenate(
        [mu, jnp.zeros((_B, 1, _SL), jnp.float32)], axis=1)

    gg = jnp.einsum('brs,csyx->byxrc', mu9, gamma_w).reshape(_B, 81, _C)
    gb_ = jnp.einsum('brs,csyx->byxrc', mu9, beta_w).reshape(_B, 81, _C)

    scale = bn_w / jnp.sqrt(var + 1e-5)
    shift = bn_b - mean * scale

    rows_a = scale[None, None, :] * gg
    rows_b = shift[None, None, :] * gg + gb_
    bias_a = jnp.broadcast_to((scale * (1.0 + gamma_b))[None, :], (_B, _C))
    bias_b = jnp.broadcast_to((shift * (1.0 + gamma_b) + beta_b)[None, :],
                              (_B, _C))

    gcat = jnp.zeros((_B, 2 * _C, 128), jnp.float32)
    gcat = gcat.at[:, :_C, :81].set(jnp.swapaxes(rows_a, 1, 2))
    gcat = gcat.at[:, _C:, :81].set(jnp.swapaxes(rows_b, 1, 2))
    gcat = gcat.at[:, :_C, 81].set(bias_a)
    gcat = gcat.at[:, _C:, 81].set(bias_b)
    return gcat


_UH = np.zeros((_HS, _W), np.float32)
for _x in range(_W):
    _UH[_x // 4, _x] = 1.0

_MK = np.ones((2, _NFLAT), np.float32)
_MK[0, 0::_W] = 0.0
_MK[1, _W - 1::_W] = 0.0


def kernel(fp, sg, style_codes, mask_codes, bn_w, bn_b, fc_w, fc_b,
           gamma_w, gamma_b, beta_w, beta_b):
    stats = pl.pallas_call(
        _stats_kernel,
        grid=(_B, _NT),
        in_specs=[pl.BlockSpec((1, _C, _TH, _W), lambda b, t: (b, 0, t, 0))],
        out_specs=pl.BlockSpec((2, _C), lambda b, t: (0, 0)),
        out_shape=jax.ShapeDtypeStruct((2, _C), jnp.float32),
    )(fp)
    n = float(_B * _H * _W)
    mean = stats[0] / n
    var = stats[1] / n - mean * mean

    gcat = _build_tables(sg, style_codes, mask_codes, bn_w, bn_b, fc_w, fc_b,
                         gamma_w, gamma_b, beta_w, beta_b, mean, var)
    uh = jnp.asarray(_UH)
    mk = jnp.asarray(_MK)
    fp_flat = fp.reshape(_B, _C, _H * _W)

    out = pl.pallas_call(
        _main_kernel,
        grid=(_B, _NT),
        in_specs=[
            pl.BlockSpec((1, _C, _NFLAT), lambda b, t: (b, 0, t)),
            pl.BlockSpec((1, _R, _HS, _WS), lambda b, t: (b, 0, 0, 0)),
            pl.BlockSpec((1, 2 * _C, 128), lambda b, t: (b, 0, 0)),
            pl.BlockSpec((_HS, _W), lambda b, t: (0, 0)),
            pl.BlockSpec((2, _NFLAT), lambda b, t: (0, 0)),
        ],
        out_specs=pl.BlockSpec((1, _C, _NFLAT), lambda b, t: (b, 0, t)),
        out_shape=jax.ShapeDtypeStruct((_B, _C, _H * _W), jnp.float32),
        scratch_shapes=[
            pltpu.VMEM((128, _NFLAT), jnp.float32),
            pltpu.VMEM((_R + 1, _HS + 2, _W), jnp.float32),
        ],
    )(fp_flat, sg, gcat, uh, mk)
    return out.reshape(_B, _C, _H, _W)


# speedup vs baseline: 2.3066x; 2.3066x over previous
import jax
import jax.numpy as jnp
import numpy as np
from jax.experimental import pallas as pl
from jax.experimental.pallas import tpu as pltpu

_B = 2
_C = 96
_H = 224
_W = 224
_SL = 64
_R = 8
_HS = 56
_WS = 56

_TH = 56
_NT = _H // _TH
_WP = 256
_NFLAT = _TH * _WP
_UROWS = _TH // 4 + 2
_UFLAT = (_UROWS * 4) * _WP

_THS = 112
_NTS = _H // _THS


def _stats_kernel(fp_ref, out_ref):
    b = pl.program_id(0)
    t = pl.program_id(1)

    @pl.when(jnp.logical_and(b == 0, t == 0))
    def _init():
        out_ref[...] = jnp.zeros_like(out_ref)

    x = fp_ref[0]
    out_ref[0, :] += jnp.sum(x, axis=(1, 2))
    out_ref[1, :] += jnp.sum(x * x, axis=(1, 2))


def _main_kernel(fp_ref, sg_ref, gcat_ref, uh_ref, out_ref, f_ref, oc_ref):
    b = pl.program_id(0)
    t = pl.program_id(1)

    @pl.when(jnp.logical_and(b == 0, t == 0))
    def _init_f():
        f_ref[81, :] = jnp.ones((_NFLAT,), jnp.float32)
        f_ref[82:128, :] = jnp.zeros((46, _NFLAT), jnp.float32)

    @pl.when(t == 0)
    def _build_onehot():
        uh = uh_ref[...]
        zrow = jnp.zeros((1, _WP), jnp.float32)
        def store_region(jj, oj):
            ocj = jnp.dot(oj, uh, preferred_element_type=jnp.float32)
            ocp = jnp.concatenate([zrow, ocj, zrow], axis=0)
            for ti in range(_NT):
                oc_ref[jj, ti, :, :] = ocp[(_TH // 4) * ti:
                                           (_TH // 4) * ti + _UROWS]

        na = jnp.ones((_HS, _WS), jnp.float32)
        for j in range(_R - 1, -1, -1):
            mj = jnp.where(sg_ref[0, j] != 0.0, 1.0, 0.0)
            store_region(j, mj * na)
            na = na * (1.0 - mj)
        store_region(_R, na)

    rs = oc_ref[:, t, :, :]
    rx = jnp.broadcast_to(rs[:, :, None, :], (_R + 1, _UROWS, 4, _WP))
    uflat = rx.reshape(_R + 1, _UFLAT)
    base = 4 * _WP
    for dy in range(3):
        for dx in range(3):
            tap = dy * 3 + dx
            st = base + (dy - 1) * _WP + (dx - 1)
            f_ref[tap * 9:(tap + 1) * 9, :] = uflat[:, st:st + _NFLAT]

    gb = jnp.dot(gcat_ref[0], f_ref[...],
                 preferred_element_type=jnp.float32)
    gb = gb.reshape(2 * _C, _TH, _WP)[:, :, :_W]
    x = fp_ref[0]
    out_ref[0] = x * gb[:_C] + gb[_C:]


def _build_tables(sg, style_codes, mask_codes, bn_w, bn_b, fc_w, fc_b,
                  gamma_w, gamma_b, beta_w, beta_b, mean, var):
    use = (mask_codes[:, :_R] == 1)[:, :, None]
    codes = jnp.where(use, style_codes[:, :_R], style_codes[:, _R:_R + 1])
    mu = jax.nn.relu(jnp.einsum('brs,rts->brt', codes, fc_w) + fc_b[None])
    mu9 = jnp.concatenate(
        [mu, jnp.zeros((_B, 1, _SL), jnp.float32)], axis=1)

    gg = jnp.einsum('brs,csyx->byxrc', mu9, gamma_w).reshape(_B, 81, _C)
    gb_ = jnp.einsum('brs,csyx->byxrc', mu9, beta_w).reshape(_B, 81, _C)

    scale = bn_w / jnp.sqrt(var + 1e-5)
    shift = bn_b - mean * scale

    rows_a = scale[None, None, :] * gg
    rows_b = shift[None, None, :] * gg + gb_
    bias_a = jnp.broadcast_to((scale * (1.0 + gamma_b))[None, :], (_B, _C))
    bias_b = jnp.broadcast_to((shift * (1.0 + gamma_b) + beta_b)[None, :],
                              (_B, _C))

    gcat = jnp.zeros((_B, 2 * _C, 128), jnp.float32)
    gcat = gcat.at[:, :_C, :81].set(jnp.swapaxes(rows_a, 1, 2))
    gcat = gcat.at[:, _C:, :81].set(jnp.swapaxes(rows_b, 1, 2))
    gcat = gcat.at[:, :_C, 81].set(bias_a)
    gcat = gcat.at[:, _C:, 81].set(bias_b)
    return gcat


_UH = np.zeros((_HS, _WP), np.float32)
for _x in range(_W):
    _UH[_x // 4, _x] = 1.0


def kernel(fp, sg, style_codes, mask_codes, bn_w, bn_b, fc_w, fc_b,
           gamma_w, gamma_b, beta_w, beta_b):
    stats = pl.pallas_call(
        _stats_kernel,
        grid=(_B, _NTS),
        in_specs=[pl.BlockSpec((1, _C, _THS, _W), lambda b, t: (b, 0, t, 0))],
        out_specs=pl.BlockSpec((2, _C), lambda b, t: (0, 0)),
        out_shape=jax.ShapeDtypeStruct((2, _C), jnp.float32),
    )(fp)
    n = float(_B * _H * _W)
    mean = stats[0] / n
    var = stats[1] / n - mean * mean

    gcat = _build_tables(sg, style_codes, mask_codes, bn_w, bn_b, fc_w, fc_b,
                         gamma_w, gamma_b, beta_w, beta_b, mean, var)
    uh = jnp.asarray(_UH)

    out = pl.pallas_call(
        _main_kernel,
        grid=(_B, _NT),
        in_specs=[
            pl.BlockSpec((1, _C, _TH, _W), lambda b, t: (b, 0, t, 0)),
            pl.BlockSpec((1, _R, _HS, _WS), lambda b, t: (b, 0, 0, 0)),
            pl.BlockSpec((1, 2 * _C, 128), lambda b, t: (b, 0, 0)),
            pl.BlockSpec((_HS, _WP), lambda b, t: (0, 0)),
        ],
        out_specs=pl.BlockSpec((1, _C, _TH, _W), lambda b, t: (b, 0, t, 0)),
        out_shape=jax.ShapeDtypeStruct((_B, _C, _H, _W), jnp.float32),
        scratch_shapes=[
            pltpu.VMEM((128, _NFLAT), jnp.float32),
            pltpu.VMEM((_R + 1, _NT, _UROWS, _WP), jnp.float32),
        ],
    )(fp, sg, gcat, uh)
    return out
